# baseline (speedup 1.0000x reference)
import functools

import jax
import jax.numpy as jnp
from jax import lax
from jax.experimental import pallas as pl
from jax.experimental.pallas import tpu as pltpu
from jax.experimental.pallas import tpu_sc as plsc

_SEQ_LENS = (2048, 1792, 1536, 1280, 1024, 768, 512, 256)
_D = 1024
_MAXL = 2048
_NC = 2
_NW = 32
_ZROWS = 32

_ZCHUNKS = tuple(i * _MAXL + r for i, L in enumerate(_SEQ_LENS)
                 for r in range(L, _MAXL, _ZROWS))
_ZPW = len(_ZCHUNKS) // _NW

_TCH = 256
_TNB = 6
_CCHUNKS = tuple((i, k * _TCH) for i, L in enumerate(_SEQ_LENS)
                 for k in range(L // _TCH))


def _zero_body(zsrc, out, zbuf, zsem):
    w = lax.axis_index("s") * _NC + lax.axis_index("c")
    pltpu.sync_copy(zsrc, zbuf)
    for j in range(_ZPW):
        for i in range(_NW):
            base = _ZCHUNKS[i * _ZPW + j]

            @pl.when(w == i)
            def _(base=base):
                pltpu.async_copy(zbuf, out.at[pl.ds(base, _ZROWS), :], zsem)
    for j in range(_ZPW):
        for i in range(_NW):
            base = _ZCHUNKS[i * _ZPW + j]

            @pl.when(w == i)
            def _(base=base):
                pltpu.make_async_copy(zbuf, out.at[pl.ds(base, _ZROWS), :],
                                      zsem).wait()


@functools.partial(
    pl.kernel,
    out_type=jax.ShapeDtypeStruct((8 * _MAXL, _D), jnp.float32),
    mesh=plsc.VectorSubcoreMesh(core_axis_name="c", subcore_axis_name="s"),
    scratch_types=[
        pltpu.VMEM((_ZROWS, _D), jnp.float32),
        pltpu.SemaphoreType.DMA,
    ],
)
def _zero_sc(*refs):
    _zero_body(*refs)


def _copy_tc_body(x0, x1, x2, x3, x4, x5, x6, x7, out_in, out, sems):
    del out_in
    xs = (x0, x1, x2, x3, x4, x5, x6, x7)
    for i, L in enumerate(_SEQ_LENS):
        pltpu.make_async_copy(xs[i], out.at[pl.ds(i * _MAXL, L), :],
                              sems.at[i]).start()
    for i, L in enumerate(_SEQ_LENS):
        pltpu.make_async_copy(xs[i], out.at[pl.ds(i * _MAXL, L), :],
                              sems.at[i]).wait()


_copy_tc = pl.pallas_call(
    _copy_tc_body,
    out_shape=jax.ShapeDtypeStruct((8 * _MAXL, _D), jnp.float32),
    in_specs=[pl.BlockSpec(memory_space=pl.ANY)] * 9,
    out_specs=pl.BlockSpec(memory_space=pl.ANY),
    input_output_aliases={8: 0},
    scratch_shapes=[pltpu.SemaphoreType.DMA((8,))],
)


def kernel(x0, x1, x2, x3, x4, x5, x6, x7):
    zsrc = jnp.zeros((_ZROWS, _D), jnp.float32)
    zeroed = _zero_sc(zsrc)
    out = _copy_tc(x0, x1, x2, x3, x4, x5, x6, x7, zeroed)
    return out.reshape(8, _MAXL, _D)


# speedup vs baseline: 19.3017x; 19.3017x over previous
import functools

import jax
import jax.numpy as jnp
from jax import lax
from jax.experimental import pallas as pl
from jax.experimental.pallas import tpu as pltpu
from jax.experimental.pallas import tpu_sc as plsc

_SEQ_LENS = (2048, 1792, 1536, 1280, 1024, 768, 512, 256)
_D = 1024
_MAXL = 2048
_NC = 2
_NW = 32
_ZROWS = 32

_ZCHUNKS = tuple(i * _MAXL + r for i, L in enumerate(_SEQ_LENS)
                 for r in range(L, _MAXL, _ZROWS))
_ZPW = len(_ZCHUNKS) // _NW

_TCH = 256
_TNB = 6
_CCHUNKS = tuple((i, k * _TCH) for i, L in enumerate(_SEQ_LENS)
                 for k in range(L // _TCH))


def _zero_body(zsrc, out, zbuf, zsem):
    w = lax.axis_index("s") * _NC + lax.axis_index("c")
    pltpu.sync_copy(zsrc, zbuf)
    for j in range(_ZPW):
        for i in range(_NW):
            base = _ZCHUNKS[i * _ZPW + j]

            @pl.when(w == i)
            def _(base=base):
                pltpu.async_copy(zbuf, out.at[pl.ds(base, _ZROWS), :], zsem)
    for j in range(_ZPW):
        for i in range(_NW):
            base = _ZCHUNKS[i * _ZPW + j]

            @pl.when(w == i)
            def _(base=base):
                pltpu.make_async_copy(zbuf, out.at[pl.ds(base, _ZROWS), :],
                                      zsem).wait()


@functools.partial(
    pl.kernel,
    out_type=jax.ShapeDtypeStruct((8 * _MAXL, _D), jnp.float32),
    mesh=plsc.VectorSubcoreMesh(core_axis_name="c", subcore_axis_name="s"),
    scratch_types=[
        pltpu.VMEM((_ZROWS, _D), jnp.float32),
        pltpu.SemaphoreType.DMA,
    ],
)
def _zero_sc(*refs):
    _zero_body(*refs)


def _copy_tc_body(x0, x1, x2, x3, x4, x5, x6, x7, out_in, out, *scratch):
    del out_in
    n = len(_CCHUNKS)
    bufs = scratch[:n]
    rsems, wsems = scratch[n], scratch[n + 1]
    xs = (x0, x1, x2, x3, x4, x5, x6, x7)

    def rd(k):
        seq, r0 = _CCHUNKS[k]
        return pltpu.make_async_copy(xs[seq].at[pl.ds(r0, _TCH), :], bufs[k],
                                     rsems.at[k])

    def wr(k):
        seq, r0 = _CCHUNKS[k]
        return pltpu.make_async_copy(bufs[k],
                                     out.at[pl.ds(seq * _MAXL + r0, _TCH), :],
                                     wsems.at[k])

    for k in range(n):
        rd(k).start()
    for k in range(n):
        rd(k).wait()
        wr(k).start()
    for k in range(n):
        wr(k).wait()


_N_CCH = len(_CCHUNKS)
_copy_tc = pl.pallas_call(
    _copy_tc_body,
    out_shape=jax.ShapeDtypeStruct((8 * _MAXL, _D), jnp.float32),
    in_specs=[pl.BlockSpec(memory_space=pl.ANY)] * 9,
    out_specs=pl.BlockSpec(memory_space=pl.ANY),
    input_output_aliases={8: 0},
    scratch_shapes=[pltpu.VMEM((_TCH, _D), jnp.float32)] * _N_CCH
    + [pltpu.SemaphoreType.DMA((_N_CCH,)),
       pltpu.SemaphoreType.DMA((_N_CCH,))],
)


def kernel(x0, x1, x2, x3, x4, x5, x6, x7):
    zsrc = jnp.zeros((_ZROWS, _D), jnp.float32)
    zeroed = _zero_sc(zsrc)
    out = _copy_tc(x0, x1, x2, x3, x4, x5, x6, x7, zeroed)
    return out.reshape(8, _MAXL, _D)


# speedup vs baseline: 19.5421x; 1.0125x over previous
import functools

import jax
import jax.numpy as jnp
from jax import lax
from jax.experimental import pallas as pl
from jax.experimental.pallas import tpu as pltpu
from jax.experimental.pallas import tpu_sc as plsc

_SEQ_LENS = (2048, 1792, 1536, 1280, 1024, 768, 512, 256)
_D = 1024
_MAXL = 2048
_NC = 2
_NW = 32
_ZROWS = 32

_ZCHUNKS = tuple(i * _MAXL + r for i, L in enumerate(_SEQ_LENS)
                 for r in range(L, _MAXL, _ZROWS))
_ZPW = len(_ZCHUNKS) // _NW

_TCH = 256
_TNB = 6
_CCHUNKS = tuple((i, k * _TCH) for i, L in enumerate(_SEQ_LENS)
                 for k in range(L // _TCH))


_ZCUM = tuple(128 * i * (i - 1) for i in range(9))


def _zero_body(zsrc, out, zbuf, zsem):
    w = lax.axis_index("s") * _NC + lax.axis_index("c")
    pltpu.sync_copy(zsrc, zbuf)
    for j in range(_ZPW):
        g = w * (_ZPW * _ZROWS) + j * _ZROWS
        i = jnp.int32(1)
        for k in range(2, 8):
            i = i + (g >= _ZCUM[k]).astype(jnp.int32)
        base = pl.multiple_of(2048 + g + 1920 * i - 128 * i * i, _ZROWS)
        pltpu.async_copy(zbuf, out.at[pl.ds(base, _ZROWS), :], zsem)
    for j in range(_ZPW):
        pltpu.make_async_copy(zbuf, out.at[pl.ds(0, _ZROWS), :], zsem).wait()


@functools.partial(
    pl.kernel,
    out_type=jax.ShapeDtypeStruct((8 * _MAXL, _D), jnp.float32),
    mesh=plsc.VectorSubcoreMesh(core_axis_name="c", subcore_axis_name="s"),
    scratch_types=[
        pltpu.VMEM((_ZROWS, _D), jnp.float32),
        pltpu.SemaphoreType.DMA,
    ],
)
def _zero_sc(*refs):
    _zero_body(*refs)


def _copy_tc_body(x0, x1, x2, x3, x4, x5, x6, x7, out_in, out, *scratch):
    del out_in
    n = len(_CCHUNKS)
    bufs = scratch[:n]
    rsems, wsems = scratch[n], scratch[n + 1]
    xs = (x0, x1, x2, x3, x4, x5, x6, x7)

    def rd(k):
        seq, r0 = _CCHUNKS[k]
        return pltpu.make_async_copy(xs[seq].at[pl.ds(r0, _TCH), :], bufs[k],
                                     rsems.at[k])

    def wr(k):
        seq, r0 = _CCHUNKS[k]
        return pltpu.make_async_copy(bufs[k],
                                     out.at[pl.ds(seq * _MAXL + r0, _TCH), :],
                                     wsems.at[k])

    for k in range(n):
        rd(k).start()
    for k in range(n):
        rd(k).wait()
        wr(k).start()
    for k in range(n):
        wr(k).wait()


_N_CCH = len(_CCHUNKS)
_copy_tc = pl.pallas_call(
    _copy_tc_body,
    out_shape=jax.ShapeDtypeStruct((8 * _MAXL, _D), jnp.float32),
    in_specs=[pl.BlockSpec(memory_space=pl.ANY)] * 9,
    out_specs=pl.BlockSpec(memory_space=pl.ANY),
    input_output_aliases={8: 0},
    scratch_shapes=[pltpu.VMEM((_TCH, _D), jnp.float32)] * _N_CCH
    + [pltpu.SemaphoreType.DMA((_N_CCH,)),
       pltpu.SemaphoreType.DMA((_N_CCH,))],
)


def kernel(x0, x1, x2, x3, x4, x5, x6, x7):
    zsrc = jnp.zeros((_ZROWS, _D), jnp.float32)
    zeroed = _zero_sc(zsrc)
    out = _copy_tc(x0, x1, x2, x3, x4, x5, x6, x7, zeroed)
    return out.reshape(8, _MAXL, _D)
